# baseline (device time: 263547 ns/iter reference)
import jax
import jax.numpy as jnp
from jax import lax
from jax.experimental import pallas as pl
from jax.experimental.pallas import tpu as pltpu

T_LOC = 1024
T_GLB = 2048
D = 1024
F = 4096
E_LOC = 8
E_GLB = 16
K = 2
P = T_GLB * K
C = 320
F_T = 1024
N_F = F // F_T


def _exchange_and_route(x_shard, r_shard):

    def body(x_ref, r_ref, xbf_ref, g_ref, rg_s, xsend_s,
             send_sems, recv_sems):
        mx = lax.axis_index("x")
        my = lax.axis_index("y")
        mz = lax.axis_index("z")
        partner = (1 - mx, my, mz)

        bar = pltpu.get_barrier_semaphore()
        pl.semaphore_signal(
            bar, inc=1, device_id=partner, device_id_type=pl.DeviceIdType.MESH
        )
        pl.semaphore_wait(bar, 1)

        def exchange_from(slot):
            rr = pltpu.make_async_remote_copy(
                src_ref=r_ref,
                dst_ref=rg_s.at[slot],
                send_sem=send_sems.at[0],
                recv_sem=recv_sems.at[0],
                device_id=partner,
                device_id_type=pl.DeviceIdType.MESH,
            )
            rr.start()
            xsend_s[...] = x_ref[...].astype(jnp.bfloat16)
            rx = pltpu.make_async_remote_copy(
                src_ref=xsend_s,
                dst_ref=xbf_ref.at[slot],
                send_sem=send_sems.at[1],
                recv_sem=recv_sems.at[1],
                device_id=partner,
                device_id_type=pl.DeviceIdType.MESH,
            )
            rx.start()
            xbf_ref[slot] = xsend_s[...]
            rg_s[slot] = r_ref[...]
            rr.wait()
            r_full = jnp.concatenate([rg_s[0], rg_s[1]], axis=1)
            g_ref[slot] = jnp.dot(
                x_ref[...], r_full,
                preferred_element_type=jnp.float32,
                precision=lax.Precision.HIGHEST,
            )
            rg = pltpu.make_async_remote_copy(
                src_ref=g_ref.at[slot],
                dst_ref=g_ref.at[slot],
                send_sem=send_sems.at[2],
                recv_sem=recv_sems.at[2],
                device_id=partner,
                device_id_type=pl.DeviceIdType.MESH,
            )
            rg.start()
            rg.wait()
            rx.wait()

        @pl.when(mx == 0)
        def _():
            exchange_from(0)

        @pl.when(mx == 1)
        def _():
            exchange_from(1)

    return pl.pallas_call(
        body,
        out_shape=(
            jax.ShapeDtypeStruct((2, T_LOC, D), jnp.bfloat16),
            jax.ShapeDtypeStruct((2, T_LOC, E_GLB), jnp.float32),
        ),
        in_specs=[
            pl.BlockSpec(memory_space=pltpu.VMEM),
            pl.BlockSpec(memory_space=pltpu.VMEM),
        ],
        out_specs=(
            pl.BlockSpec(memory_space=pltpu.VMEM),
            pl.BlockSpec(memory_space=pltpu.VMEM),
        ),
        scratch_shapes=[
            pltpu.VMEM((2, D, E_LOC), jnp.float32),
            pltpu.VMEM((T_LOC, D), jnp.bfloat16),
            pltpu.SemaphoreType.DMA((3,)),
            pltpu.SemaphoreType.DMA((3,)),
        ],
        compiler_params=pltpu.CompilerParams(collective_id=0),
    )(x_shard, r_shard)


def _moe_fused(xg, pair_e, pair_slot, pair_twpad, w1, w2):

    def body(
        x_ref, pe_ref, ps_ref, twp_ref, w1_ref, w2_ref, o_ref,
        xbf_s, mp_s, mask_s, xg_s, w_s, y_s, part_s, send_bf, recv_bf,
        send_sem, recv_sem,
    ):
        e = pl.program_id(0)
        f = pl.program_id(1)
        mx = lax.axis_index("x")
        my = lax.axis_index("y")
        mz = lax.axis_index("z")
        partner = (1 - mx, my, mz)

        @pl.when((e == 0) & (f == 0))
        def _():
            bar = pltpu.get_barrier_semaphore()
            pl.semaphore_signal(
                bar, inc=1, device_id=partner,
                device_id_type=pl.DeviceIdType.MESH,
            )
            pl.semaphore_wait(bar, 1)
            xbf_s[pl.ds(0, T_LOC), :] = x_ref[0]
            xbf_s[pl.ds(T_LOC, T_LOC), :] = x_ref[1]

        @pl.when(f == 0)
        def _():
            eid = mx * E_LOC + e
            ci = lax.broadcasted_iota(jnp.int32, (C, P), 0)
            mp_s[...] = jnp.where(
                (pe_ref[...] == eid) & (ps_ref[...] == ci), 1.0, 0.0
            )
            twc = jnp.dot(
                mp_s[...], twp_ref[...],
                preferred_element_type=jnp.float32,
                precision=lax.Precision.HIGHEST,
            )
            tok = twc[:, 0:1].astype(jnp.int32)
            w_s[...] = twc[:, 1:2]
            ids = lax.broadcasted_iota(jnp.int32, (C, T_GLB), 1)
            mask_s[...] = (tok == ids).astype(jnp.bfloat16)
            xg_s[...] = jnp.dot(
                mask_s[...], xbf_s[...], preferred_element_type=jnp.float32
            ).astype(jnp.bfloat16)

        h = jnp.maximum(
            jnp.dot(
                xg_s[...],
                w1_ref[0].astype(jnp.bfloat16),
                preferred_element_type=jnp.float32,
            ),
            0.0,
        ).astype(jnp.bfloat16)
        y = jnp.dot(
            h, w2_ref[0].astype(jnp.bfloat16),
            preferred_element_type=jnp.float32,
        )

        @pl.when(f == 0)
        def _():
            y_s[...] = y

        @pl.when(f != 0)
        def _():
            y_s[...] = y_s[...] + y

        @pl.when(f == N_F - 1)
        def _():
            yw = (y_s[...] * w_s[...]).astype(jnp.bfloat16)
            contrib = lax.dot_general(
                mask_s[...], yw,
                (((0,), (0,)), ((), ())),
                preferred_element_type=jnp.float32,
            )

            @pl.when(e == 0)
            def _():
                part_s[...] = contrib

            @pl.when(e != 0)
            def _():
                part_s[...] = part_s[...] + contrib

        @pl.when((e == E_LOC - 1) & (f == N_F - 1))
        def _():
            def combine(half):
                send_bf[...] = part_s[
                    pl.ds((1 - half) * T_LOC, T_LOC), :
                ].astype(jnp.bfloat16)
                rdma = pltpu.make_async_remote_copy(
                    src_ref=send_bf,
                    dst_ref=recv_bf,
                    send_sem=send_sem,
                    recv_sem=recv_sem,
                    device_id=partner,
                    device_id_type=pl.DeviceIdType.MESH,
                )
                rdma.start()
                rdma.wait()
                o_ref[...] = (
                    part_s[pl.ds(half * T_LOC, T_LOC), :]
                    + recv_bf[...].astype(jnp.float32)
                )

            @pl.when(mx == 0)
            def _():
                combine(0)

            @pl.when(mx == 1)
            def _():
                combine(1)

    return pl.pallas_call(
        body,
        grid=(E_LOC, N_F),
        in_specs=[
            pl.BlockSpec((2, T_LOC, D), lambda e, f: (0, 0, 0)),
            pl.BlockSpec((1, P), lambda e, f: (0, 0)),
            pl.BlockSpec((1, P), lambda e, f: (0, 0)),
            pl.BlockSpec((P, 128), lambda e, f: (0, 0)),
            pl.BlockSpec((1, D, F_T), lambda e, f: (e, 0, f)),
            pl.BlockSpec((1, F_T, D), lambda e, f: (e, f, 0)),
        ],
        out_specs=pl.BlockSpec((T_LOC, D), lambda e, f: (0, 0)),
        out_shape=jax.ShapeDtypeStruct((T_LOC, D), jnp.float32),
        scratch_shapes=[
            pltpu.VMEM((T_GLB, D), jnp.bfloat16),
            pltpu.VMEM((C, P), jnp.float32),
            pltpu.VMEM((C, T_GLB), jnp.bfloat16),
            pltpu.VMEM((C, D), jnp.bfloat16),
            pltpu.VMEM((C, 1), jnp.float32),
            pltpu.VMEM((C, D), jnp.float32),
            pltpu.VMEM((T_GLB, D), jnp.float32),
            pltpu.VMEM((T_LOC, D), jnp.bfloat16),
            pltpu.VMEM((T_LOC, D), jnp.bfloat16),
            pltpu.SemaphoreType.DMA,
            pltpu.SemaphoreType.DMA,
        ],
        compiler_params=pltpu.CompilerParams(
            collective_id=1, vmem_limit_bytes=64 * 1024 * 1024
        ),
    )(xg, pair_e, pair_slot, pair_twpad, w1, w2)


def kernel(x, router, W1, W2):
    xg, g = _exchange_and_route(x, router)
    gates = g.reshape(T_GLB, E_GLB)

    top_v, top_i = lax.top_k(gates, K)
    top_w = jnp.exp(top_v - top_v.max(axis=1, keepdims=True))
    top_w = top_w / top_w.sum(axis=1, keepdims=True)

    e_flat = top_i.reshape(-1).astype(jnp.int32)
    t_flat = jnp.repeat(jnp.arange(T_GLB, dtype=jnp.int32), K)
    w_flat = top_w.reshape(-1)

    oh = (e_flat[:, None] == jnp.arange(E_GLB, dtype=jnp.int32)[None, :])
    oh = oh.astype(jnp.int32)
    slot = (jnp.cumsum(oh, axis=0) * oh).sum(axis=1) - 1

    pair_e = e_flat.reshape(1, P)
    pair_slot = slot.reshape(1, P)
    pair_twpad = jnp.concatenate(
        [
            t_flat.astype(jnp.float32)[:, None],
            w_flat[:, None],
            jnp.zeros((P, 126), jnp.float32),
        ],
        axis=1,
    )

    return _moe_fused(xg, pair_e, pair_slot, pair_twpad, W1, W2)


# device time: 216775 ns/iter; 1.2158x vs baseline; 1.2158x over previous
import jax
import jax.numpy as jnp
from jax import lax
from jax.experimental import pallas as pl
from jax.experimental.pallas import tpu as pltpu

T_LOC = 1024
T_GLB = 2048
D = 1024
F = 4096
E_LOC = 8
E_GLB = 16
K = 2
P = T_GLB * K
C = 320
F_T = 1024
N_F = F // F_T


def _exchange_and_route(x_shard, r_shard):

    def body(x_ref, r_ref, xbf_ref, g_ref, rg_s, xsend_s,
             send_sems, recv_sems):
        mx = lax.axis_index("x")
        my = lax.axis_index("y")
        mz = lax.axis_index("z")
        partner = (1 - mx, my, mz)

        bar = pltpu.get_barrier_semaphore()
        pl.semaphore_signal(
            bar, inc=1, device_id=partner, device_id_type=pl.DeviceIdType.MESH
        )
        pl.semaphore_wait(bar, 1)

        def exchange_from(slot):
            rr = pltpu.make_async_remote_copy(
                src_ref=r_ref,
                dst_ref=rg_s.at[slot],
                send_sem=send_sems.at[0],
                recv_sem=recv_sems.at[0],
                device_id=partner,
                device_id_type=pl.DeviceIdType.MESH,
            )
            rr.start()
            xsend_s[...] = x_ref[...].astype(jnp.bfloat16)
            rx = pltpu.make_async_remote_copy(
                src_ref=xsend_s,
                dst_ref=xbf_ref.at[slot],
                send_sem=send_sems.at[1],
                recv_sem=recv_sems.at[1],
                device_id=partner,
                device_id_type=pl.DeviceIdType.MESH,
            )
            rx.start()
            xbf_ref[slot] = xsend_s[...]
            rg_s[slot] = r_ref[...]
            rr.wait()
            r_full = jnp.concatenate([rg_s[0], rg_s[1]], axis=1)
            g_ref[slot] = jnp.dot(
                x_ref[...], r_full,
                preferred_element_type=jnp.float32,
                precision=lax.Precision.HIGHEST,
            )
            rg = pltpu.make_async_remote_copy(
                src_ref=g_ref.at[slot],
                dst_ref=g_ref.at[slot],
                send_sem=send_sems.at[2],
                recv_sem=recv_sems.at[2],
                device_id=partner,
                device_id_type=pl.DeviceIdType.MESH,
            )
            rg.start()
            rg.wait()
            rx.wait()

        @pl.when(mx == 0)
        def _():
            exchange_from(0)

        @pl.when(mx == 1)
        def _():
            exchange_from(1)

    return pl.pallas_call(
        body,
        out_shape=(
            jax.ShapeDtypeStruct((2, T_LOC, D), jnp.bfloat16),
            jax.ShapeDtypeStruct((2, T_LOC, E_GLB), jnp.float32),
        ),
        in_specs=[
            pl.BlockSpec(memory_space=pltpu.VMEM),
            pl.BlockSpec(memory_space=pltpu.VMEM),
        ],
        out_specs=(
            pl.BlockSpec(memory_space=pltpu.VMEM),
            pl.BlockSpec(memory_space=pltpu.VMEM),
        ),
        scratch_shapes=[
            pltpu.VMEM((2, D, E_LOC), jnp.float32),
            pltpu.VMEM((T_LOC, D), jnp.bfloat16),
            pltpu.SemaphoreType.DMA((3,)),
            pltpu.SemaphoreType.DMA((3,)),
        ],
        compiler_params=pltpu.CompilerParams(collective_id=0),
    )(x_shard, r_shard)


def _moe_fused(xg, pair_e, pair_slot, pair_t, pair_w, w1, w2):

    def body(
        x_ref, pe_ref, ps_ref, pt_ref, pw_ref, w1_ref, w2_ref, o_ref,
        xbf_s, mask_s, xg_s, w_s, y_s, part_s, send_bf, recv_bf,
        send_sem, recv_sem,
    ):
        e = pl.program_id(0)
        f = pl.program_id(1)
        mx = lax.axis_index("x")
        my = lax.axis_index("y")
        mz = lax.axis_index("z")
        partner = (1 - mx, my, mz)

        @pl.when((e == 0) & (f == 0))
        def _():
            bar = pltpu.get_barrier_semaphore()
            pl.semaphore_signal(
                bar, inc=1, device_id=partner,
                device_id_type=pl.DeviceIdType.MESH,
            )
            pl.semaphore_wait(bar, 1)
            xbf_s[pl.ds(0, T_LOC), :] = x_ref[0]
            xbf_s[pl.ds(T_LOC, T_LOC), :] = x_ref[1]

        @pl.when(f == 0)
        def _():
            eid = mx * E_LOC + e
            ci = lax.broadcasted_iota(jnp.int32, (C, P), 0)
            m_pair = jnp.where(
                (pe_ref[...] == eid) & (ps_ref[...] == ci), 1.0, 0.0
            )
            tok = jnp.sum(
                m_pair * pt_ref[...], axis=1, keepdims=True
            ).astype(jnp.int32)
            w_s[...] = jnp.sum(m_pair * pw_ref[...], axis=1, keepdims=True)
            ids = lax.broadcasted_iota(jnp.int32, (C, T_GLB), 1)
            mask_s[...] = (tok == ids).astype(jnp.bfloat16)
            xg_s[...] = jnp.dot(
                mask_s[...], xbf_s[...], preferred_element_type=jnp.float32
            ).astype(jnp.bfloat16)

        h = jnp.maximum(
            jnp.dot(
                xg_s[...],
                w1_ref[0].astype(jnp.bfloat16),
                preferred_element_type=jnp.float32,
            ),
            0.0,
        ).astype(jnp.bfloat16)
        y = jnp.dot(
            h, w2_ref[0].astype(jnp.bfloat16),
            preferred_element_type=jnp.float32,
        )

        @pl.when(f == 0)
        def _():
            y_s[...] = y

        @pl.when(f != 0)
        def _():
            y_s[...] = y_s[...] + y

        @pl.when(f == N_F - 1)
        def _():
            yw = (y_s[...] * w_s[...]).astype(jnp.bfloat16)
            contrib = lax.dot_general(
                mask_s[...], yw,
                (((0,), (0,)), ((), ())),
                preferred_element_type=jnp.float32,
            )

            @pl.when(e == 0)
            def _():
                part_s[...] = contrib

            @pl.when(e != 0)
            def _():
                part_s[...] = part_s[...] + contrib

        @pl.when((e == E_LOC - 1) & (f == N_F - 1))
        def _():
            def combine(half):
                send_bf[...] = part_s[
                    pl.ds((1 - half) * T_LOC, T_LOC), :
                ].astype(jnp.bfloat16)
                rdma = pltpu.make_async_remote_copy(
                    src_ref=send_bf,
                    dst_ref=recv_bf,
                    send_sem=send_sem,
                    recv_sem=recv_sem,
                    device_id=partner,
                    device_id_type=pl.DeviceIdType.MESH,
                )
                rdma.start()
                rdma.wait()
                o_ref[...] = (
                    part_s[pl.ds(half * T_LOC, T_LOC), :]
                    + recv_bf[...].astype(jnp.float32)
                )

            @pl.when(mx == 0)
            def _():
                combine(0)

            @pl.when(mx == 1)
            def _():
                combine(1)

    return pl.pallas_call(
        body,
        grid=(E_LOC, N_F),
        in_specs=[
            pl.BlockSpec((2, T_LOC, D), lambda e, f: (0, 0, 0)),
            pl.BlockSpec((1, P), lambda e, f: (0, 0)),
            pl.BlockSpec((1, P), lambda e, f: (0, 0)),
            pl.BlockSpec((1, P), lambda e, f: (0, 0)),
            pl.BlockSpec((1, P), lambda e, f: (0, 0)),
            pl.BlockSpec((1, D, F_T), lambda e, f: (e, 0, f)),
            pl.BlockSpec((1, F_T, D), lambda e, f: (e, f, 0)),
        ],
        out_specs=pl.BlockSpec((T_LOC, D), lambda e, f: (0, 0)),
        out_shape=jax.ShapeDtypeStruct((T_LOC, D), jnp.float32),
        scratch_shapes=[
            pltpu.VMEM((T_GLB, D), jnp.bfloat16),
            pltpu.VMEM((C, T_GLB), jnp.bfloat16),
            pltpu.VMEM((C, D), jnp.bfloat16),
            pltpu.VMEM((C, 1), jnp.float32),
            pltpu.VMEM((C, D), jnp.float32),
            pltpu.VMEM((T_GLB, D), jnp.float32),
            pltpu.VMEM((T_LOC, D), jnp.bfloat16),
            pltpu.VMEM((T_LOC, D), jnp.bfloat16),
            pltpu.SemaphoreType.DMA,
            pltpu.SemaphoreType.DMA,
        ],
        compiler_params=pltpu.CompilerParams(
            collective_id=1, vmem_limit_bytes=64 * 1024 * 1024
        ),
    )(xg, pair_e, pair_slot, pair_t, pair_w, w1, w2)


def kernel(x, router, W1, W2):
    xg, g = _exchange_and_route(x, router)
    gates = g.reshape(T_GLB, E_GLB)

    top_v, top_i = lax.top_k(gates, K)
    top_w = jnp.exp(top_v - top_v.max(axis=1, keepdims=True))
    top_w = top_w / top_w.sum(axis=1, keepdims=True)

    e_flat = top_i.reshape(-1).astype(jnp.int32)
    t_flat = jnp.repeat(jnp.arange(T_GLB, dtype=jnp.int32), K)
    w_flat = top_w.reshape(-1)

    oh = (e_flat[:, None] == jnp.arange(E_GLB, dtype=jnp.int32)[None, :])
    oh = oh.astype(jnp.int32)
    slot = (jnp.cumsum(oh, axis=0) * oh).sum(axis=1) - 1

    pair_e = e_flat.reshape(1, P)
    pair_slot = slot.reshape(1, P)
    pair_t = t_flat.astype(jnp.float32).reshape(1, P)
    pair_w = w_flat.reshape(1, P)

    return _moe_fused(xg, pair_e, pair_slot, pair_t, pair_w, W1, W2)
